# baseline (speedup 1.0000x reference)
import jax
import jax.numpy as jnp
from jax.experimental import pallas as pl
from jax.experimental.pallas import tpu as pltpu


TM = 400


def _gcn_kernel(x_ref, adj_ref, w1_ref, b1_ref, w2_ref, b2_ref,
                out_ref, s1_ref, s2_ref):
    i = pl.program_id(0)
    T = pl.num_programs(0) // 2

    @pl.when(i == 0)
    def _():
        s1_ref[...] = jnp.dot(x_ref[...], w1_ref[...],
                              preferred_element_type=jnp.float32)

    @pl.when(i < T)
    def _():
        z = jnp.dot(adj_ref[...], s1_ref[...],
                    preferred_element_type=jnp.float32) + b1_ref[...]
        h = jnp.maximum(z, 0.0)
        s2_ref[pl.ds(i * TM, TM), :] = jnp.dot(
            h, w2_ref[...], preferred_element_type=jnp.float32)
        out_ref[...] = jnp.zeros_like(out_ref)

    @pl.when(i >= T)
    def _():
        out_ref[...] = jnp.dot(adj_ref[...], s2_ref[...],
                               preferred_element_type=jnp.float32) + b2_ref[...]


def kernel(x, adj, W1, b1, W2, b2):
    n, nfeat = x.shape
    nhid = W1.shape[1]
    nclass = W2.shape[1]
    T = n // TM

    b1r = b1.reshape(1, nhid)
    b2r = b2.reshape(1, nclass)

    return pl.pallas_call(
        _gcn_kernel,
        grid=(2 * T,),
        in_specs=[
            pl.BlockSpec((n, nfeat), lambda i: (0, 0)),
            pl.BlockSpec((TM, n), lambda i: (i % T, 0)),
            pl.BlockSpec((nfeat, nhid), lambda i: (0, 0)),
            pl.BlockSpec((1, nhid), lambda i: (0, 0)),
            pl.BlockSpec((nhid, nclass), lambda i: (0, 0)),
            pl.BlockSpec((1, nclass), lambda i: (0, 0)),
        ],
        out_specs=pl.BlockSpec((TM, nclass), lambda i: (i % T, 0)),
        out_shape=jax.ShapeDtypeStruct((n, nclass), jnp.float32),
        scratch_shapes=[
            pltpu.VMEM((n, nhid), jnp.float32),
            pltpu.VMEM((n, nclass), jnp.float32),
        ],
    )(x, adj, W1, b1r, W2, b2r)


# speedup vs baseline: 1.0283x; 1.0283x over previous
import numpy as np
import jax
import jax.numpy as jnp
from jax.experimental import pallas as pl
from jax.experimental.pallas import tpu as pltpu


TM = 1024
NCACHE = 13

_ADJ_R, _ADJ_C, _CMP_R, _CMP_C, _KIND, _LOWER, _SLOT, _FIN = range(8)


def _build_schedule(T: int, ncache: int) -> np.ndarray:
    steps = []
    for r in range(T):
        for c in [j for j in range(T) if j != r] + [r]:
            steps.append([r, c, r, c, 1 if c == r else 0,
                          1 if c < r else 0, -1, 0])
    uppers = [(r, c) for r in range(T) for c in range(r + 1, T)]
    slot_of = {t: k for k, t in enumerate(uppers[:ncache])}
    for s in steps:
        key = (s[_CMP_R], s[_CMP_C])
        if s[_KIND] == 0 and key in slot_of:
            s[_SLOT] = slot_of[key]
    for (r, c) in uppers:
        if (r, c) in slot_of:
            steps.append([T - 1, T - 1, r, c, 2, 0, slot_of[(r, c)], 0])
    for (r, c) in uppers:
        if (r, c) not in slot_of:
            steps.append([r, c, r, c, 2, 0, -1, 0])
    last_touch = {}
    for idx, s in enumerate(steps):
        last_touch[s[_CMP_R]] = idx
    for idx in last_touch.values():
        steps[idx][_FIN] = 1
    return np.asarray(steps, dtype=np.int32).T.copy()


def _gcn_body(meta_ref, x_ref, adj_ref, w1_ref, b1_ref, w2_ref, b2_ref,
              out_ref, s12_ref, z_ref, oacc_ref, cache_ref):
    i = pl.program_id(0)
    n, nfeat = x_ref.shape
    nhid = w1_ref.shape[1]
    nclass = w2_ref.shape[1]
    r = meta_ref[_CMP_R, i]
    c = meta_ref[_CMP_C, i]
    kind = meta_ref[_KIND, i]
    lower = meta_ref[_LOWER, i]
    slot = meta_ref[_SLOT, i]
    fin = meta_ref[_FIN, i]
    bf = jnp.bfloat16

    @pl.when(i == 0)
    def _():
        s12_ref[...] = jnp.zeros_like(s12_ref)
        z_ref[...] = jnp.zeros_like(z_ref)
        oacc_ref[...] = jnp.zeros_like(oacc_ref)
        s1 = jnp.dot(x_ref[...].astype(bf), w1_ref[...].astype(bf),
                     preferred_element_type=jnp.float32)
        s12_ref[0:n, 0:nhid] = s1.astype(bf)

    @pl.when(kind < 2)
    def _():
        a_bf = adj_ref[...].astype(bf)
        p = jnp.dot(a_bf, s12_ref[pl.ds(c * TM, TM), :],
                    preferred_element_type=jnp.float32)
        z_ref[...] += p[:, :nhid]

        @pl.when(lower == 1)
        def _():
            oacc_ref[pl.ds(r * TM, TM), :] += p[:, nhid:]

        @pl.when(slot >= 0)
        def _():
            cache_ref[pl.ds(slot * TM, TM), :] = a_bf

        @pl.when(kind == 1)
        def _():
            h = jnp.maximum(z_ref[...] + b1_ref[...], 0.0)
            rows = jax.lax.broadcasted_iota(jnp.int32, h.shape, 0)
            h = jnp.where(rows < n - r * TM, h, 0.0).astype(bf)
            s2b = jnp.dot(h, w2_ref[...].astype(bf),
                          preferred_element_type=jnp.float32).astype(bf)
            s12_ref[pl.ds(r * TM, TM), nhid:] = s2b
            oacc_ref[pl.ds(r * TM, TM), :] += jnp.dot(
                a_bf, s2b, preferred_element_type=jnp.float32)
            z_ref[...] = jnp.zeros_like(z_ref)

    @pl.when(kind == 2)
    def _():
        s2c = s12_ref[pl.ds(c * TM, TM), nhid:]

        @pl.when(slot >= 0)
        def _():
            oacc_ref[pl.ds(r * TM, TM), :] += jnp.dot(
                cache_ref[pl.ds(slot * TM, TM), :], s2c,
                preferred_element_type=jnp.float32)

        @pl.when(slot < 0)
        def _():
            oacc_ref[pl.ds(r * TM, TM), :] += jnp.dot(
                adj_ref[...].astype(bf), s2c,
                preferred_element_type=jnp.float32)

    @pl.when(fin == 1)
    def _():
        oacc_ref[pl.ds(r * TM, TM), :] += b2_ref[...]

    @pl.when(i == pl.num_programs(0) - 1)
    def _():
        out_ref[...] = oacc_ref[0:n, :]


def kernel(x, adj, W1, b1, W2, b2):
    n, nfeat = x.shape
    nhid = W1.shape[1]
    nclass = W2.shape[1]
    T = -(-n // TM)
    npad = T * TM
    meta = jnp.asarray(_build_schedule(T, NCACHE))
    G = meta.shape[1]

    grid_spec = pltpu.PrefetchScalarGridSpec(
        num_scalar_prefetch=1,
        grid=(G,),
        in_specs=[
            pl.BlockSpec((n, nfeat), lambda i, m: (0, 0)),
            pl.BlockSpec((TM, TM), lambda i, m: (m[_ADJ_R, i], m[_ADJ_C, i])),
            pl.BlockSpec((nfeat, nhid), lambda i, m: (0, 0)),
            pl.BlockSpec((1, nhid), lambda i, m: (0, 0)),
            pl.BlockSpec((nhid, nclass), lambda i, m: (0, 0)),
            pl.BlockSpec((1, nclass), lambda i, m: (0, 0)),
        ],
        out_specs=pl.BlockSpec((n, nclass), lambda i, m: (0, 0)),
        scratch_shapes=[
            pltpu.VMEM((npad, nhid + nclass), jnp.bfloat16),
            pltpu.VMEM((TM, nhid), jnp.float32),
            pltpu.VMEM((npad, nclass), jnp.float32),
            pltpu.VMEM((NCACHE * TM, TM), jnp.bfloat16),
        ],
    )
    return pl.pallas_call(
        _gcn_body,
        grid_spec=grid_spec,
        out_shape=jax.ShapeDtypeStruct((n, nclass), jnp.float32),
    )(meta, x, adj, W1, b1.reshape(1, nhid), W2, b2.reshape(1, nclass))


# speedup vs baseline: 1.0384x; 1.0098x over previous
import numpy as np
import jax
import jax.numpy as jnp
from jax.experimental import pallas as pl
from jax.experimental.pallas import tpu as pltpu


TM = 1024
NCACHE = 13

_ADJ_R, _ADJ_C, _CMP_R, _CMP_C, _KIND, _LOWER, _SLOT, _FIN = range(8)


def _build_schedule(T: int, ncache: int) -> np.ndarray:
    steps = []
    for r in range(T):
        for c in [j for j in range(T) if j != r] + [r]:
            steps.append([r, c, r, c, 1 if c == r else 0,
                          1 if c < r else 0, -1, 0])
    uppers = [(r, c) for r in range(T) for c in range(r + 1, T)]
    slot_of = {t: k for k, t in enumerate(uppers[:ncache])}
    for s in steps:
        key = (s[_CMP_R], s[_CMP_C])
        if s[_KIND] == 0 and key in slot_of:
            s[_SLOT] = slot_of[key]
    for (r, c) in uppers:
        if (r, c) in slot_of:
            steps.append([T - 1, T - 1, r, c, 2, 0, slot_of[(r, c)], 0])
    for (r, c) in uppers:
        if (r, c) not in slot_of:
            steps.append([r, c, r, c, 2, 0, -1, 0])
    last_touch = {}
    for idx, s in enumerate(steps):
        last_touch[s[_CMP_R]] = idx
    for idx in last_touch.values():
        steps[idx][_FIN] = 1
    return np.asarray(steps, dtype=np.int32).T.copy()


def _gcn_body(meta_ref, x_ref, adj_ref, w1_ref, b1_ref, w2_ref, b2_ref,
              out_ref, s12_ref, s12b_ref, z_ref, oacc_ref, cache_ref):
    i = pl.program_id(0)
    n, nfeat = x_ref.shape
    nhid = w1_ref.shape[1]
    nclass = w2_ref.shape[1]
    r = meta_ref[_CMP_R, i]
    c = meta_ref[_CMP_C, i]
    kind = meta_ref[_KIND, i]
    lower = meta_ref[_LOWER, i]
    slot = meta_ref[_SLOT, i]
    fin = meta_ref[_FIN, i]
    bf = jnp.bfloat16

    @pl.when(i == 0)
    def _():
        s12_ref[...] = jnp.zeros_like(s12_ref)
        s12b_ref[...] = jnp.zeros_like(s12b_ref)
        z_ref[...] = jnp.zeros_like(z_ref)
        oacc_ref[...] = jnp.zeros_like(oacc_ref)
        s1 = jax.lax.dot_general(
            x_ref[...], w1_ref[...], (((1,), (0,)), ((), ())),
            precision=jax.lax.Precision.DEFAULT,
            preferred_element_type=jnp.float32)
        s12_ref[0:n, 0:nhid] = s1
        s12b_ref[0:n, 0:nhid] = s1.astype(bf)

    @pl.when(kind < 2)
    def _():
        p = jax.lax.dot_general(
            adj_ref[...], s12_ref[pl.ds(c * TM, TM), :],
            (((1,), (0,)), ((), ())),
            precision=jax.lax.Precision.DEFAULT,
            preferred_element_type=jnp.float32)
        z_ref[...] += p[:, :nhid]

        @pl.when(lower == 1)
        def _():
            oacc_ref[pl.ds(r * TM, TM), :] += p[:, nhid:]

        @pl.when(slot >= 0)
        def _():
            cache_ref[pl.ds(slot * TM, TM), :] = adj_ref[...].astype(bf)

        @pl.when(kind == 1)
        def _():
            h = jnp.maximum(z_ref[...] + b1_ref[...], 0.0)
            rows = jax.lax.broadcasted_iota(jnp.int32, h.shape, 0)
            h = jnp.where(rows < n - r * TM, h, 0.0)
            s2b = jax.lax.dot_general(
                h, w2_ref[...], (((1,), (0,)), ((), ())),
                precision=jax.lax.Precision.DEFAULT,
                preferred_element_type=jnp.float32)
            s12_ref[pl.ds(r * TM, TM), nhid:] = s2b
            s12b_ref[pl.ds(r * TM, TM), nhid:] = s2b.astype(bf)
            oacc_ref[pl.ds(r * TM, TM), :] += jax.lax.dot_general(
                adj_ref[...], s2b, (((1,), (0,)), ((), ())),
                precision=jax.lax.Precision.DEFAULT,
                preferred_element_type=jnp.float32)
            z_ref[...] = jnp.zeros_like(z_ref)

    @pl.when(kind == 2)
    def _():
        @pl.when(slot >= 0)
        def _():
            oacc_ref[pl.ds(r * TM, TM), :] += jnp.dot(
                cache_ref[pl.ds(slot * TM, TM), :],
                s12b_ref[pl.ds(c * TM, TM), nhid:],
                preferred_element_type=jnp.float32)

        @pl.when(slot < 0)
        def _():
            oacc_ref[pl.ds(r * TM, TM), :] += jax.lax.dot_general(
                adj_ref[...], s12_ref[pl.ds(c * TM, TM), nhid:],
                (((1,), (0,)), ((), ())),
                precision=jax.lax.Precision.DEFAULT,
                preferred_element_type=jnp.float32)

    @pl.when(fin == 1)
    def _():
        oacc_ref[pl.ds(r * TM, TM), :] += b2_ref[...]

    @pl.when(i == pl.num_programs(0) - 1)
    def _():
        out_ref[...] = oacc_ref[0:n, :]


def kernel(x, adj, W1, b1, W2, b2):
    n, nfeat = x.shape
    nhid = W1.shape[1]
    nclass = W2.shape[1]
    T = -(-n // TM)
    npad = T * TM
    meta = jnp.asarray(_build_schedule(T, NCACHE))
    G = meta.shape[1]

    grid_spec = pltpu.PrefetchScalarGridSpec(
        num_scalar_prefetch=1,
        grid=(G,),
        in_specs=[
            pl.BlockSpec((n, nfeat), lambda i, m: (0, 0)),
            pl.BlockSpec((TM, TM), lambda i, m: (m[_ADJ_R, i], m[_ADJ_C, i])),
            pl.BlockSpec((nfeat, nhid), lambda i, m: (0, 0)),
            pl.BlockSpec((1, nhid), lambda i, m: (0, 0)),
            pl.BlockSpec((nhid, nclass), lambda i, m: (0, 0)),
            pl.BlockSpec((1, nclass), lambda i, m: (0, 0)),
        ],
        out_specs=pl.BlockSpec((n, nclass), lambda i, m: (0, 0)),
        scratch_shapes=[
            pltpu.VMEM((npad, nhid + nclass), jnp.float32),
            pltpu.VMEM((npad, nhid + nclass), jnp.bfloat16),
            pltpu.VMEM((TM, nhid), jnp.float32),
            pltpu.VMEM((npad, nclass), jnp.float32),
            pltpu.VMEM((NCACHE * TM, TM), jnp.bfloat16),
        ],
    )
    return pl.pallas_call(
        _gcn_body,
        grid_spec=grid_spec,
        out_shape=jax.ShapeDtypeStruct((n, nclass), jnp.float32),
    )(meta, x, adj, W1, b1.reshape(1, nhid), W2, b2.reshape(1, nclass))
